# Initial kernel scaffold; baseline (speedup 1.0000x reference)
import functools

import jax
import jax.numpy as jnp
from jax import lax
from jax.experimental import pallas as pl
from jax.experimental.pallas import tpu as pltpu
from jax.experimental.pallas import tpu_sc as plsc

NC = 2
NS = 16
NW = NC * NS
LANES = 16


def _make_spmm(n, d, e, dtype):
    epw = e // NW
    chunk = 80
    nchunk = epw // chunk
    rps = n // NS
    mesh = plsc.VectorSubcoreMesh(core_axis_name="c", subcore_axis_name="s")

    @functools.partial(
        pl.kernel,
        mesh=mesh,
        out_type=jax.ShapeDtypeStruct((NC, n, d), dtype),
        scratch_types=[
            pltpu.VMEM_SHARED((n, d), dtype),
            pltpu.VMEM((chunk,), jnp.int32),
            pltpu.VMEM((chunk,), jnp.int32),
            pltpu.VMEM((chunk,), dtype),
            pltpu.VMEM((chunk, d), dtype),
            pltpu.SemaphoreType.DMA,
        ],
    )
    def spmm(h_hbm, src_hbm, dst_hbm, lv_hbm, z_hbm, out_hbm,
             acc, src_v, dst_v, lv_v, rows_v, sem):
        cid = lax.axis_index("c")
        sid = lax.axis_index("s")
        wid = sid * NC + cid

        pltpu.sync_copy(z_hbm, acc.at[pl.ds(sid * rps, rps)])
        plsc.subcore_barrier()

        def body(i, carry):
            base = wid * epw + i * chunk
            pltpu.sync_copy(src_hbm.at[pl.ds(base, chunk)], src_v)
            pltpu.sync_copy(dst_hbm.at[pl.ds(base, chunk)], dst_v)
            pltpu.sync_copy(lv_hbm.at[pl.ds(base, chunk)], lv_v)
            pltpu.async_copy(h_hbm.at[src_v], rows_v, sem).wait()

            def scale(eidx, c2):
                w = lv_v[eidx]
                for j in range(d // LANES):
                    sl = pl.ds(j * LANES, LANES)
                    rows_v[eidx, sl] = rows_v[eidx, sl] * w
                return c2

            lax.fori_loop(0, chunk, scale, 0)
            pltpu.sync_copy(rows_v, acc.at[dst_v], add=True)
            return carry

        lax.fori_loop(0, nchunk, body, 0)
        plsc.subcore_barrier()
        pltpu.sync_copy(acc.at[pl.ds(sid * rps, rps)],
                        out_hbm.at[cid, pl.ds(sid * rps, rps)])

    return spmm


def _dense_layer(h, p0, p1, w_cat, b):
    n, d = h.shape
    rblk = 1000
    grid = (n // rblk,)

    def body(h_ref, p0_ref, p1_ref, w_ref, b_ref, hn_ref, nrm_ref):
        lh = p0_ref[...] + p1_ref[...]
        hv = h_ref[...]
        cat = jnp.concatenate([lh + hv, hv * lh], axis=1)
        y = jnp.dot(cat, w_ref[...], preferred_element_type=jnp.float32)
        y = y + b_ref[...]
        y = jnp.where(y >= 0, y, 0.2 * y)
        hn_ref[...] = y
        ss = jnp.sum(y * y, axis=1, keepdims=True)
        nrm_ref[...] = y * lax.rsqrt(jnp.maximum(ss, 1e-12))

    row_spec = pl.BlockSpec((rblk, d), lambda i: (i, 0))
    return pl.pallas_call(
        body,
        grid=grid,
        in_specs=[
            row_spec, row_spec, row_spec,
            pl.BlockSpec((2 * d, d), lambda i: (0, 0)),
            pl.BlockSpec((1, d), lambda i: (0, 0)),
        ],
        out_specs=[row_spec, row_spec],
        out_shape=[
            jax.ShapeDtypeStruct((n, d), h.dtype),
            jax.ShapeDtypeStruct((n, d), h.dtype),
        ],
    )(h, p0, p1, w_cat, b)


def kernel(x, edge_index, L_vals, W_gc, b_gc, W_bi, b_bi):
    n, d = x.shape
    e = L_vals.shape[0]
    k = W_gc.shape[0]
    src = edge_index[0]
    dst = edge_index[1]
    zeros = jnp.zeros((n // NS, d), x.dtype)
    spmm = _make_spmm(n, d, e, x.dtype)

    h = x
    outs = [x]
    for i in range(k):
        p = spmm(h, src, dst, L_vals, zeros)
        w_cat = jnp.concatenate([W_gc[i], W_bi[i]], axis=0)
        b = (b_gc[i] + b_bi[i]).reshape(1, d)
        h, nrm = _dense_layer(h, p[0], p[1], w_cat, b)
        outs.append(nrm)
    return jnp.concatenate(outs, axis=1)


# speedup vs baseline: 3.9525x; 3.9525x over previous
import functools

import jax
import jax.numpy as jnp
from jax import lax
from jax.experimental import pallas as pl
from jax.experimental.pallas import tpu as pltpu
from jax.experimental.pallas import tpu_sc as plsc

NC = 2
NS = 16
NW = NC * NS
LANES = 16


def _make_spmm(n, d, e, dtype):
    epw = e // NW
    chunk = 80
    nchunk = epw // chunk
    rps = (n // NS) // 8 * 8
    rem = n - rps * NS
    mesh = plsc.VectorSubcoreMesh(core_axis_name="c", subcore_axis_name="s")

    @functools.partial(
        pl.kernel,
        mesh=mesh,
        out_type=jax.ShapeDtypeStruct((NC, n, d), dtype),
        scratch_types=[
            pltpu.VMEM_SHARED((n, d), dtype),
            pltpu.VMEM((chunk,), jnp.int32),
            pltpu.VMEM((chunk,), jnp.int32),
            pltpu.VMEM((chunk,), dtype),
            pltpu.VMEM((chunk, d), dtype),
            pltpu.SemaphoreType.DMA,
        ],
    )
    def spmm(h_hbm, src_hbm, dst_hbm, lv_hbm, z_hbm, out_hbm,
             acc, src_v, dst_v, lv_v, rows_v, sem):
        cid = lax.axis_index("c")
        sid = lax.axis_index("s")
        wid = sid * NC + cid

        pltpu.sync_copy(z_hbm.at[pl.ds(0, rps)], acc.at[pl.ds(sid * rps, rps)])
        if rem:
            @pl.when(sid == NS - 1)
            def _zero_tail():
                pltpu.sync_copy(z_hbm.at[pl.ds(0, rem)],
                                acc.at[pl.ds(rps * NS, rem)])
        plsc.subcore_barrier()

        def body(i, carry):
            base = wid * epw + i * chunk
            pltpu.sync_copy(src_hbm.at[pl.ds(base, chunk)], src_v)
            pltpu.sync_copy(dst_hbm.at[pl.ds(base, chunk)], dst_v)
            pltpu.sync_copy(lv_hbm.at[pl.ds(base, chunk)], lv_v)
            pltpu.async_copy(h_hbm.at[src_v], rows_v, sem).wait()

            def scale(g, c2):
                w16 = lv_v[pl.ds(g * LANES, LANES)]
                for jj in range(LANES):
                    wj = w16[jj]
                    row = g * LANES + jj
                    for j in range(d // LANES):
                        sl = pl.ds(j * LANES, LANES)
                        rows_v[row, sl] = rows_v[row, sl] * wj
                return c2

            lax.fori_loop(0, chunk // LANES, scale, 0)
            pltpu.sync_copy(rows_v, acc.at[dst_v], add=True)
            return carry

        lax.fori_loop(0, nchunk, body, 0)
        plsc.subcore_barrier()
        pltpu.sync_copy(acc.at[pl.ds(sid * rps, rps)],
                        out_hbm.at[cid, pl.ds(sid * rps, rps)])
        if rem:
            @pl.when(sid == NS - 1)
            def _out_tail():
                pltpu.sync_copy(acc.at[pl.ds(rps * NS, rem)],
                                out_hbm.at[cid, pl.ds(rps * NS, rem)])

    return spmm


def _dense_layer(h, p0, p1, w_cat, b):
    n, d = h.shape
    rblk = 1000
    grid = (n // rblk,)

    def body(h_ref, p0_ref, p1_ref, w_ref, b_ref, hn_ref, nrm_ref):
        lh = p0_ref[...] + p1_ref[...]
        hv = h_ref[...]
        cat = jnp.concatenate([lh + hv, hv * lh], axis=1)
        y = jnp.dot(cat, w_ref[...], preferred_element_type=jnp.float32)
        y = y + b_ref[...]
        y = jnp.where(y >= 0, y, 0.2 * y)
        hn_ref[...] = y
        ss = jnp.sum(y * y, axis=1, keepdims=True)
        nrm_ref[...] = y * lax.rsqrt(jnp.maximum(ss, 1e-12))

    row_spec = pl.BlockSpec((rblk, d), lambda i: (i, 0))
    return pl.pallas_call(
        body,
        grid=grid,
        in_specs=[
            row_spec, row_spec, row_spec,
            pl.BlockSpec((2 * d, d), lambda i: (0, 0)),
            pl.BlockSpec((1, d), lambda i: (0, 0)),
        ],
        out_specs=[row_spec, row_spec],
        out_shape=[
            jax.ShapeDtypeStruct((n, d), h.dtype),
            jax.ShapeDtypeStruct((n, d), h.dtype),
        ],
    )(h, p0, p1, w_cat, b)


def kernel(x, edge_index, L_vals, W_gc, b_gc, W_bi, b_bi):
    n, d = x.shape
    e = L_vals.shape[0]
    k = W_gc.shape[0]
    src = edge_index[0]
    dst = edge_index[1]
    zeros = jnp.zeros(((n // NS) // 8 * 8, d), x.dtype)
    spmm = _make_spmm(n, d, e, x.dtype)

    h = x
    outs = [x]
    for i in range(k):
        p = spmm(h, src, dst, L_vals, zeros)
        w_cat = jnp.concatenate([W_gc[i], W_bi[i]], axis=0)
        b = (b_gc[i] + b_bi[i]).reshape(1, d)
        h, nrm = _dense_layer(h, p[0], p[1], w_cat, b)
        outs.append(nrm)
    return jnp.concatenate(outs, axis=1)


# speedup vs baseline: 9.9514x; 2.5177x over previous
import functools

import jax
import jax.numpy as jnp
from jax import lax
from jax.experimental import pallas as pl
from jax.experimental.pallas import tpu as pltpu
from jax.experimental.pallas import tpu_sc as plsc

NC = 2
NS = 16
NW = NC * NS
LANES = 16


def _make_spmm(n, d, e, dtype):
    epw = e // NW
    chunk = 80
    nchunk = epw // chunk
    rps = (n // NS) // 8 * 8
    rem = n - rps * NS
    mesh = plsc.VectorSubcoreMesh(core_axis_name="c", subcore_axis_name="s")

    assert nchunk % 2 == 1 and nchunk >= 3

    @functools.partial(
        pl.kernel,
        mesh=mesh,
        out_type=jax.ShapeDtypeStruct((NC, n, d), dtype),
        scratch_types=[
            pltpu.VMEM_SHARED((n, d), dtype),
            pltpu.VMEM((2, chunk), jnp.int32),
            pltpu.VMEM((2, chunk), jnp.int32),
            pltpu.VMEM((2, chunk), dtype),
            pltpu.VMEM((chunk, d), dtype),
            pltpu.VMEM((chunk, d), dtype),
            pltpu.SemaphoreType.DMA((2,)),
            pltpu.SemaphoreType.DMA((2,)),
            pltpu.SemaphoreType.DMA((2,)),
            pltpu.SemaphoreType.DMA((2,)),
        ],
    )
    def spmm(h_hbm, src_hbm, dst_hbm, lv_hbm, z_hbm, out_hbm,
             acc, src_v, dst_v, lv_v, rows0, rows1,
             isem, jsem, gsem, ssem):
        cid = lax.axis_index("c")
        sid = lax.axis_index("s")
        wid = sid * NC + cid
        ebase = wid * epw
        rows = (rows0, rows1)

        pltpu.sync_copy(z_hbm.at[pl.ds(0, rps)], acc.at[pl.ds(sid * rps, rps)])
        if rem:
            @pl.when(sid == NS - 1)
            def _zero_tail():
                pltpu.sync_copy(z_hbm.at[pl.ds(0, rem)],
                                acc.at[pl.ds(rps * NS, rem)])
        plsc.subcore_barrier()

        def scale(rows_b, lv_ref):
            def grp(g, c2):
                w16 = lv_ref[pl.ds(g * LANES, LANES)]
                for jj in range(LANES):
                    wj = w16[jj]
                    row = g * LANES + jj
                    for j in range(d // LANES):
                        sl = pl.ds(j * LANES, LANES)
                        rows_b[row, sl] = rows_b[row, sl] * wj
                return c2
            lax.fori_loop(0, chunk // LANES, grp, 0)

        def gather_issue(i, b):
            pltpu.async_copy(h_hbm.at[src_v.at[b]], rows[b], gsem.at[b])

        def scatter_issue(b):
            pltpu.async_copy(rows[b], acc.at[dst_v.at[b]], ssem.at[b],
                             add=True)

        pltpu.sync_copy(src_hbm.at[pl.ds(ebase, chunk)], src_v.at[0])
        gather_issue(0, 0)
        pltpu.async_copy(src_hbm.at[pl.ds(ebase + chunk, chunk)],
                         src_v.at[1], isem.at[1])
        pltpu.async_copy(dst_hbm.at[pl.ds(ebase, chunk)], dst_v.at[0],
                         jsem.at[0])
        pltpu.async_copy(lv_hbm.at[pl.ds(ebase, chunk)], lv_v.at[0],
                         jsem.at[0])

        def wait_scatter(b):
            pltpu.make_async_copy(rows[b], acc.at[dst_v.at[b]],
                                  ssem.at[b]).wait()

        def wait_gather(b):
            pltpu.make_async_copy(h_hbm.at[src_v.at[b]], rows[b],
                                  gsem.at[b]).wait()

        def wait_src(i, b):
            off = ebase + i * chunk
            pltpu.make_async_copy(src_hbm.at[pl.ds(off, chunk)],
                                  src_v.at[b], isem.at[b]).wait()

        def wait_dstlv(i, b):
            off = ebase + i * chunk
            pltpu.make_async_copy(dst_hbm.at[pl.ds(off, chunk)],
                                  dst_v.at[b], jsem.at[b]).wait()
            pltpu.make_async_copy(lv_hbm.at[pl.ds(off, chunk)],
                                  lv_v.at[b], jsem.at[b]).wait()

        def body_steps(i, b, prefetch_next_jl, prefetch_src2, last):
            nb = 1 - b
            pred = i >= 1
            if isinstance(pred, bool):
                if pred:
                    wait_scatter(nb)
            else:
                @pl.when(pred)
                def _w_scatter():
                    wait_scatter(nb)
            if prefetch_next_jl:
                off = ebase + (i + 1) * chunk
                pltpu.async_copy(dst_hbm.at[pl.ds(off, chunk)],
                                 dst_v.at[nb], jsem.at[nb])
                pltpu.async_copy(lv_hbm.at[pl.ds(off, chunk)],
                                 lv_v.at[nb], jsem.at[nb])
            if not last:
                wait_src(i + 1, nb)
                gather_issue(i + 1, nb)
            wait_gather(b)
            if prefetch_src2 is not None:
                @pl.when(prefetch_src2)
                def _pf_src():
                    off2 = ebase + (i + 2) * chunk
                    pltpu.async_copy(src_hbm.at[pl.ds(off2, chunk)],
                                     src_v.at[b], isem.at[b])
            wait_dstlv(i, b)
            scale(rows[b], lv_v.at[b])
            scatter_issue(b)

        def pair(i2, carry):
            i = 2 * i2
            body_steps(i, 0, True, i + 2 < nchunk, False)
            body_steps(i + 1, 1, True, i + 3 < nchunk, False)
            return carry

        lax.fori_loop(0, (nchunk - 1) // 2, pair, 0)
        body_steps(nchunk - 1, 0, False, None, True)
        wait_scatter(0)
        plsc.subcore_barrier()
        pltpu.sync_copy(acc.at[pl.ds(sid * rps, rps)],
                        out_hbm.at[cid, pl.ds(sid * rps, rps)])
        if rem:
            @pl.when(sid == NS - 1)
            def _out_tail():
                pltpu.sync_copy(acc.at[pl.ds(rps * NS, rem)],
                                out_hbm.at[cid, pl.ds(rps * NS, rem)])

    return spmm


def _dense_layer(h, p0, p1, w_cat, b):
    n, d = h.shape
    rblk = 1000
    grid = (n // rblk,)

    def body(h_ref, p0_ref, p1_ref, w_ref, b_ref, hn_ref, nrm_ref):
        lh = p0_ref[...] + p1_ref[...]
        hv = h_ref[...]
        cat = jnp.concatenate([lh + hv, hv * lh], axis=1)
        y = jnp.dot(cat, w_ref[...], preferred_element_type=jnp.float32)
        y = y + b_ref[...]
        y = jnp.where(y >= 0, y, 0.2 * y)
        hn_ref[...] = y
        ss = jnp.sum(y * y, axis=1, keepdims=True)
        nrm_ref[...] = y * lax.rsqrt(jnp.maximum(ss, 1e-12))

    row_spec = pl.BlockSpec((rblk, d), lambda i: (i, 0))
    return pl.pallas_call(
        body,
        grid=grid,
        in_specs=[
            row_spec, row_spec, row_spec,
            pl.BlockSpec((2 * d, d), lambda i: (0, 0)),
            pl.BlockSpec((1, d), lambda i: (0, 0)),
        ],
        out_specs=[row_spec, row_spec],
        out_shape=[
            jax.ShapeDtypeStruct((n, d), h.dtype),
            jax.ShapeDtypeStruct((n, d), h.dtype),
        ],
    )(h, p0, p1, w_cat, b)


def kernel(x, edge_index, L_vals, W_gc, b_gc, W_bi, b_bi):
    n, d = x.shape
    e = L_vals.shape[0]
    k = W_gc.shape[0]
    src = edge_index[0]
    dst = edge_index[1]
    zeros = jnp.zeros(((n // NS) // 8 * 8, d), x.dtype)
    spmm = _make_spmm(n, d, e, x.dtype)

    h = x
    outs = [x]
    for i in range(k):
        p = spmm(h, src, dst, L_vals, zeros)
        w_cat = jnp.concatenate([W_gc[i], W_bi[i]], axis=0)
        b = (b_gc[i] + b_bi[i]).reshape(1, d)
        h, nrm = _dense_layer(h, p[0], p[1], w_cat, b)
        outs.append(nrm)
    return jnp.concatenate(outs, axis=1)


# speedup vs baseline: 10.9139x; 1.0967x over previous
import functools

import jax
import jax.numpy as jnp
from jax import lax
from jax.experimental import pallas as pl
from jax.experimental.pallas import tpu as pltpu
from jax.experimental.pallas import tpu_sc as plsc

NC = 2
NS = 16
NW = NC * NS
LANES = 16


def _make_spmm(n, d, e, dtype):
    epw = e // NW
    chunk = 80
    nchunk = epw // chunk
    rps = (n // NS) // 8 * 8
    rem = n - rps * NS
    mesh = plsc.VectorSubcoreMesh(core_axis_name="c", subcore_axis_name="s")

    BUF = 3
    H = next(h for h in range(BUF - 1, 3 * BUF)
             if (nchunk - 3 - h) % BUF == 0 and nchunk - 3 - h >= 0)
    G = (nchunk - 3 - H) // BUF

    @functools.partial(
        pl.kernel,
        mesh=mesh,
        out_type=jax.ShapeDtypeStruct((NC, n, d), dtype),
        scratch_types=[
            pltpu.VMEM_SHARED((n, d), dtype),
            pltpu.VMEM((BUF, chunk), jnp.int32),
            pltpu.VMEM((BUF, chunk), jnp.int32),
            pltpu.VMEM((BUF, chunk), dtype),
            pltpu.VMEM((chunk, d), dtype),
            pltpu.VMEM((chunk, d), dtype),
            pltpu.VMEM((chunk, d), dtype),
            pltpu.SemaphoreType.DMA((BUF,)),
            pltpu.SemaphoreType.DMA((BUF,)),
            pltpu.SemaphoreType.DMA((BUF,)),
            pltpu.SemaphoreType.DMA((BUF,)),
        ],
    )
    def spmm(h_hbm, src_hbm, dst_hbm, lv_hbm, z_hbm, out_hbm,
             acc, src_v, dst_v, lv_v, rows0, rows1, rows2,
             isem, jsem, gsem, ssem):
        cid = lax.axis_index("c")
        sid = lax.axis_index("s")
        wid = sid * NC + cid
        ebase = wid * epw
        rows = (rows0, rows1, rows2)

        pltpu.sync_copy(z_hbm.at[pl.ds(0, rps)], acc.at[pl.ds(sid * rps, rps)])
        if rem:
            @pl.when(sid == NS - 1)
            def _zero_tail():
                pltpu.sync_copy(z_hbm.at[pl.ds(0, rem)],
                                acc.at[pl.ds(rps * NS, rem)])
        plsc.subcore_barrier()

        def scale(rows_b, lv_ref):
            def grp(g, c2):
                w16 = lv_ref[pl.ds(g * LANES, LANES)]
                for jj in range(LANES):
                    wj = w16[jj]
                    row = g * LANES + jj
                    for j in range(d // LANES):
                        sl = pl.ds(j * LANES, LANES)
                        rows_b[row, sl] = rows_b[row, sl] * wj
                return c2
            lax.fori_loop(0, chunk // LANES, grp, 0)

        def gather_issue(i, b):
            pltpu.async_copy(h_hbm.at[src_v.at[b]], rows[b], gsem.at[b])

        def scatter_issue(b):
            pltpu.async_copy(rows[b], acc.at[dst_v.at[b]], ssem.at[b],
                             add=True)

        def wait_scatter(b):
            pltpu.make_async_copy(rows[b], acc.at[dst_v.at[b]],
                                  ssem.at[b]).wait()

        def wait_gather(b):
            pltpu.make_async_copy(h_hbm.at[src_v.at[b]], rows[b],
                                  gsem.at[b]).wait()

        def issue_src(i, b):
            off = ebase + i * chunk
            pltpu.async_copy(src_hbm.at[pl.ds(off, chunk)],
                             src_v.at[b], isem.at[b])

        def wait_src(i, b):
            off = ebase + i * chunk
            pltpu.make_async_copy(src_hbm.at[pl.ds(off, chunk)],
                                  src_v.at[b], isem.at[b]).wait()

        def issue_dl(i, b):
            off = ebase + i * chunk
            pltpu.async_copy(dst_hbm.at[pl.ds(off, chunk)],
                             dst_v.at[b], jsem.at[b])
            pltpu.async_copy(lv_hbm.at[pl.ds(off, chunk)],
                             lv_v.at[b], jsem.at[b])

        def wait_dstlv(i, b):
            off = ebase + i * chunk
            pltpu.make_async_copy(dst_hbm.at[pl.ds(off, chunk)],
                                  dst_v.at[b], jsem.at[b]).wait()
            pltpu.make_async_copy(lv_hbm.at[pl.ds(off, chunk)],
                                  lv_v.at[b], jsem.at[b]).wait()

        pltpu.sync_copy(src_hbm.at[pl.ds(ebase, chunk)], src_v.at[0])
        gather_issue(0, 0)
        pltpu.sync_copy(src_hbm.at[pl.ds(ebase + chunk, chunk)],
                        src_v.at[1])
        gather_issue(1, 1)
        issue_src(2, 2)
        issue_dl(0, 0)
        issue_dl(1, 1)

        def body_steps(i, b, wait_prev=True, pf_dl=True, pf_g=True,
                       pf_src=True):
            bp = (b + BUF - 1) % BUF
            if wait_prev:
                wait_scatter(bp)
            if pf_dl:
                issue_dl(i + 2, bp)
            if pf_g:
                wait_src(i + 2, bp)
                gather_issue(i + 2, bp)
            wait_gather(b)
            if pf_src:
                issue_src(i + 3, b)
            wait_dstlv(i, b)
            scale(rows[b], lv_v.at[b])
            scatter_issue(b)

        for i in range(H):
            body_steps(i, i % BUF, wait_prev=(i >= 1))

        def group(g, carry):
            i0 = H + BUF * g
            for r in range(BUF):
                body_steps(i0 + r, (H + r) % BUF)
            return carry

        lax.fori_loop(0, G, group, 0)

        t0 = nchunk - 3
        body_steps(t0, t0 % BUF, pf_src=False)
        body_steps(t0 + 1, (t0 + 1) % BUF, pf_dl=False, pf_g=False,
                   pf_src=False)
        body_steps(t0 + 2, (t0 + 2) % BUF, pf_dl=False, pf_g=False,
                   pf_src=False)
        wait_scatter((nchunk - 1) % BUF)
        plsc.subcore_barrier()
        pltpu.sync_copy(acc.at[pl.ds(sid * rps, rps)],
                        out_hbm.at[cid, pl.ds(sid * rps, rps)])
        if rem:
            @pl.when(sid == NS - 1)
            def _out_tail():
                pltpu.sync_copy(acc.at[pl.ds(rps * NS, rem)],
                                out_hbm.at[cid, pl.ds(rps * NS, rem)])

    return spmm


def _dense_layer(h, p0, p1, w_cat, b):
    n, d = h.shape
    rblk = 1000
    grid = (n // rblk,)

    def body(h_ref, p0_ref, p1_ref, w_ref, b_ref, hn_ref, nrm_ref):
        lh = p0_ref[...] + p1_ref[...]
        hv = h_ref[...]
        cat = jnp.concatenate([lh + hv, hv * lh], axis=1)
        y = jnp.dot(cat, w_ref[...], preferred_element_type=jnp.float32)
        y = y + b_ref[...]
        y = jnp.where(y >= 0, y, 0.2 * y)
        hn_ref[...] = y
        ss = jnp.sum(y * y, axis=1, keepdims=True)
        nrm_ref[...] = y * lax.rsqrt(jnp.maximum(ss, 1e-12))

    row_spec = pl.BlockSpec((rblk, d), lambda i: (i, 0))
    return pl.pallas_call(
        body,
        grid=grid,
        in_specs=[
            row_spec, row_spec, row_spec,
            pl.BlockSpec((2 * d, d), lambda i: (0, 0)),
            pl.BlockSpec((1, d), lambda i: (0, 0)),
        ],
        out_specs=[row_spec, row_spec],
        out_shape=[
            jax.ShapeDtypeStruct((n, d), h.dtype),
            jax.ShapeDtypeStruct((n, d), h.dtype),
        ],
    )(h, p0, p1, w_cat, b)


def kernel(x, edge_index, L_vals, W_gc, b_gc, W_bi, b_bi):
    n, d = x.shape
    e = L_vals.shape[0]
    k = W_gc.shape[0]
    src = edge_index[0]
    dst = edge_index[1]
    zeros = jnp.zeros(((n // NS) // 8 * 8, d), x.dtype)
    spmm = _make_spmm(n, d, e, x.dtype)

    h = x
    outs = [x]
    for i in range(k):
        p = spmm(h, src, dst, L_vals, zeros)
        w_cat = jnp.concatenate([W_gc[i], W_bi[i]], axis=0)
        b = (b_gc[i] + b_bi[i]).reshape(1, d)
        h, nrm = _dense_layer(h, p[0], p[1], w_cat, b)
        outs.append(nrm)
    return jnp.concatenate(outs, axis=1)


# speedup vs baseline: 11.5895x; 1.0619x over previous
import functools

import jax
import jax.numpy as jnp
from jax import lax
from jax.experimental import pallas as pl
from jax.experimental.pallas import tpu as pltpu
from jax.experimental.pallas import tpu_sc as plsc

NC = 2
NS = 16
NW = NC * NS
LANES = 16


def _make_spmm(n, d, e, dtype):
    epw = e // NW
    chunk = 80
    nchunk = epw // chunk
    rps = (n // NS) // 8 * 8
    rem = n - rps * NS
    mesh = plsc.VectorSubcoreMesh(core_axis_name="c", subcore_axis_name="s")

    BUF = 3
    H = next(h for h in range(BUF - 1, 3 * BUF)
             if (nchunk - 3 - h) % BUF == 0 and nchunk - 3 - h >= 0)
    G = (nchunk - 3 - H) // BUF

    @functools.partial(
        pl.kernel,
        mesh=mesh,
        out_type=jax.ShapeDtypeStruct((NC, n, d), dtype),
        scratch_types=[
            pltpu.VMEM_SHARED((n, d), dtype),
            pltpu.VMEM((BUF, chunk), jnp.int32),
            pltpu.VMEM((BUF, chunk), jnp.int32),
            pltpu.VMEM((BUF, chunk), dtype),
            pltpu.VMEM((chunk, d), dtype),
            pltpu.VMEM((chunk, d), dtype),
            pltpu.VMEM((chunk, d), dtype),
            pltpu.SemaphoreType.DMA((BUF,)),
            pltpu.SemaphoreType.DMA((BUF,)),
            pltpu.SemaphoreType.DMA((BUF,)),
            pltpu.SemaphoreType.DMA((BUF,)),
        ],
    )
    def spmm(h_hbm, src_hbm, dst_hbm, lv_hbm, z_hbm, out_hbm,
             acc, src_v, dst_v, lv_v, rows0, rows1, rows2,
             isem, jsem, gsem, ssem):
        cid = lax.axis_index("c")
        sid = lax.axis_index("s")
        wid = sid * NC + cid
        ebase = wid * epw
        rows = (rows0, rows1, rows2)

        pltpu.sync_copy(z_hbm.at[pl.ds(0, rps)], acc.at[pl.ds(sid * rps, rps)])
        if rem:
            @pl.when(sid == NS - 1)
            def _zero_tail():
                pltpu.sync_copy(z_hbm.at[pl.ds(0, rem)],
                                acc.at[pl.ds(rps * NS, rem)])
        plsc.subcore_barrier()

        def scale(rows_b, lv_ref):
            def grp(g, c2):
                w16 = lv_ref[pl.ds(g * LANES, LANES)]
                for jj in range(LANES):
                    wj = w16[jj]
                    row = g * LANES + jj
                    for j in range(d // LANES):
                        sl = pl.ds(j * LANES, LANES)
                        rows_b[row, sl] = rows_b[row, sl] * wj
                return c2
            lax.fori_loop(0, chunk // LANES, grp, 0)

        def gather_issue(i, b):
            pltpu.async_copy(h_hbm.at[src_v.at[b]], rows[b], gsem.at[b])

        def scatter_issue(b):
            pltpu.async_copy(rows[b], acc.at[dst_v.at[b]], ssem.at[b],
                             add=True)

        def wait_scatter(b):
            pltpu.make_async_copy(rows[b], acc.at[dst_v.at[b]],
                                  ssem.at[b]).wait()

        def wait_gather(b):
            pltpu.make_async_copy(h_hbm.at[src_v.at[b]], rows[b],
                                  gsem.at[b]).wait()

        def issue_src(i, b):
            off = ebase + i * chunk
            pltpu.async_copy(src_hbm.at[pl.ds(off, chunk)],
                             src_v.at[b], isem.at[b])

        def wait_src(i, b):
            off = ebase + i * chunk
            pltpu.make_async_copy(src_hbm.at[pl.ds(off, chunk)],
                                  src_v.at[b], isem.at[b]).wait()

        def issue_dl(i, b):
            off = ebase + i * chunk
            pltpu.async_copy(dst_hbm.at[pl.ds(off, chunk)],
                             dst_v.at[b], jsem.at[b])
            pltpu.async_copy(lv_hbm.at[pl.ds(off, chunk)],
                             lv_v.at[b], jsem.at[b])

        def wait_dstlv(i, b):
            off = ebase + i * chunk
            pltpu.make_async_copy(dst_hbm.at[pl.ds(off, chunk)],
                                  dst_v.at[b], jsem.at[b]).wait()
            pltpu.make_async_copy(lv_hbm.at[pl.ds(off, chunk)],
                                  lv_v.at[b], jsem.at[b]).wait()

        pltpu.sync_copy(src_hbm.at[pl.ds(ebase, chunk)], src_v.at[0])
        gather_issue(0, 0)
        pltpu.sync_copy(src_hbm.at[pl.ds(ebase + chunk, chunk)],
                        src_v.at[1])
        gather_issue(1, 1)
        issue_src(2, 2)
        issue_dl(0, 0)
        issue_dl(1, 1)

        def body_steps(i, b, wait_prev=True, pf_dl=True, pf_g=True,
                       pf_src=True):
            bp = (b + BUF - 1) % BUF
            if wait_prev:
                wait_scatter(bp)
            if pf_dl:
                issue_dl(i + 2, bp)
            if pf_g:
                wait_src(i + 2, bp)
                gather_issue(i + 2, bp)
            wait_gather(b)
            if pf_src:
                issue_src(i + 3, b)
            wait_dstlv(i, b)
            scale(rows[b], lv_v.at[b])
            scatter_issue(b)

        for i in range(H):
            body_steps(i, i % BUF, wait_prev=(i >= 1))

        def group(g, carry):
            i0 = H + BUF * g
            for r in range(BUF):
                body_steps(i0 + r, (H + r) % BUF)
            return carry

        lax.fori_loop(0, G, group, 0)

        t0 = nchunk - 3
        body_steps(t0, t0 % BUF, pf_src=False)
        body_steps(t0 + 1, (t0 + 1) % BUF, pf_dl=False, pf_g=False,
                   pf_src=False)
        body_steps(t0 + 2, (t0 + 2) % BUF, pf_dl=False, pf_g=False,
                   pf_src=False)
        wait_scatter((nchunk - 1) % BUF)
        plsc.subcore_barrier()
        pltpu.sync_copy(acc.at[pl.ds(sid * rps, rps)],
                        out_hbm.at[cid, pl.ds(sid * rps, rps)])
        if rem:
            @pl.when(sid == NS - 1)
            def _out_tail():
                pltpu.sync_copy(acc.at[pl.ds(rps * NS, rem)],
                                out_hbm.at[cid, pl.ds(rps * NS, rem)])

    return spmm


def _dense_layer(h, p, w_cat, b):
    n, d = h.shape
    rblk = 1000
    grid = (n // rblk,)

    def body(h_ref, p_ref, w_ref, b_ref, hn_ref, nrm_ref):
        lh = p_ref[0] + p_ref[1]
        hv = h_ref[...]
        cat = jnp.concatenate([lh + hv, hv * lh], axis=1)
        y = jnp.dot(cat, w_ref[...], preferred_element_type=jnp.float32)
        y = y + b_ref[...]
        y = jnp.where(y >= 0, y, 0.2 * y)
        hn_ref[...] = y
        ss = jnp.sum(y * y, axis=1, keepdims=True)
        nrm_ref[...] = y * lax.rsqrt(jnp.maximum(ss, 1e-12))

    row_spec = pl.BlockSpec((rblk, d), lambda i: (i, 0))
    return pl.pallas_call(
        body,
        grid=grid,
        in_specs=[
            row_spec,
            pl.BlockSpec((2, rblk, d), lambda i: (0, i, 0)),
            pl.BlockSpec((2 * d, d), lambda i: (0, 0)),
            pl.BlockSpec((1, d), lambda i: (0, 0)),
        ],
        out_specs=[row_spec, row_spec],
        out_shape=[
            jax.ShapeDtypeStruct((n, d), h.dtype),
            jax.ShapeDtypeStruct((n, d), h.dtype),
        ],
    )(h, p, w_cat, b)


def _dense_layer_final(h, p, w_cat, b, prevs):
    n, d = h.shape
    rblk = 1000
    grid = (n // rblk,)
    nprev = len(prevs)
    dout = (nprev + 1) * d

    def body(h_ref, p_ref, w_ref, b_ref, *rest):
        prev_refs, out_ref = rest[:nprev], rest[nprev]
        lh = p_ref[0] + p_ref[1]
        hv = h_ref[...]
        cat = jnp.concatenate([lh + hv, hv * lh], axis=1)
        y = jnp.dot(cat, w_ref[...], preferred_element_type=jnp.float32)
        y = y + b_ref[...]
        y = jnp.where(y >= 0, y, 0.2 * y)
        ss = jnp.sum(y * y, axis=1, keepdims=True)
        nrm = y * lax.rsqrt(jnp.maximum(ss, 1e-12))
        out_ref[...] = jnp.concatenate(
            [r[...] for r in prev_refs] + [nrm], axis=1)

    row_spec = pl.BlockSpec((rblk, d), lambda i: (i, 0))
    return pl.pallas_call(
        body,
        grid=grid,
        in_specs=[
            row_spec,
            pl.BlockSpec((2, rblk, d), lambda i: (0, i, 0)),
            pl.BlockSpec((2 * d, d), lambda i: (0, 0)),
            pl.BlockSpec((1, d), lambda i: (0, 0)),
        ] + [row_spec] * nprev,
        out_specs=pl.BlockSpec((rblk, dout), lambda i: (i, 0)),
        out_shape=jax.ShapeDtypeStruct((n, dout), h.dtype),
    )(h, p, w_cat, b, *prevs)


def kernel(x, edge_index, L_vals, W_gc, b_gc, W_bi, b_bi):
    n, d = x.shape
    e = L_vals.shape[0]
    k = W_gc.shape[0]
    src = edge_index[0]
    dst = edge_index[1]
    zeros = jnp.zeros(((n // NS) // 8 * 8, d), x.dtype)
    spmm = _make_spmm(n, d, e, x.dtype)

    h = x
    nrms = []
    for i in range(k):
        p = spmm(h, src, dst, L_vals, zeros)
        w_cat = jnp.concatenate([W_gc[i], W_bi[i]], axis=0)
        b = (b_gc[i] + b_bi[i]).reshape(1, d)
        if i < k - 1:
            h, nrm = _dense_layer(h, p, w_cat, b)
            nrms.append(nrm)
        else:
            out = _dense_layer_final(h, p, w_cat, b, [x] + nrms)
    return out
